# Initial kernel scaffold; baseline (speedup 1.0000x reference)
import jax
import jax.numpy as jnp
from jax.experimental import pallas as pl


def kernel(image_features, text_features, ln_t_scale, ln_t_bias, W_t, b_t, ln_i_scale, ln_i_bias, W_i, b_i, W_tw1, b_tw1, W_tw2, b_tw2, temperature, diversity_lambda):
    raise NotImplementedError("write your pallas kernel here")


# speedup vs baseline: 1.6451x; 1.6451x over previous
import functools

import jax
import jax.numpy as jnp
from jax import lax
from jax.experimental import pallas as pl
from jax.experimental.pallas import tpu as pltpu

B, NI, NT, H, D = 16, 1024, 77, 768, 64
NUM_QUERY = 16
_SQRT_HALF = 0.7071067811865476


def _gelu(x):
    return 0.5 * x * (1.0 + lax.erf(x * _SQRT_HALF))


def _ln(x, scale, bias):
    m = x.mean(-1, keepdims=True)
    v = ((x - m) ** 2).mean(-1, keepdims=True)
    return (x - m) / jnp.sqrt(v + 1e-5) * scale + bias


def _select_body(img_ref, txt_ref, gum_ref,
                 ln_t_scale, ln_t_bias, W_t, b_t,
                 ln_i_scale, ln_i_bias, W_i, b_i,
                 W_tw1, b_tw1, W_tw2, b_tw2,
                 temp_ref, lam_ref, sel_ref):
    t = txt_ref[0]
    x = img_ref[0]
    g = gum_ref[0]

    tc = _gelu(_ln(t, ln_t_scale[0], ln_t_bias[0]) @ W_t[...] + b_t[0]) + t[:, :D]
    tw = _gelu(t @ W_tw1[...] + b_tw1[0]) @ W_tw2[...] + b_tw2[0]
    sg = jax.nn.sigmoid(tw)
    sg_m = sg - jnp.max(sg, axis=0, keepdims=True)
    e_t = jnp.exp(sg_m)
    w = e_t / jnp.sum(e_t, axis=0, keepdims=True)
    wt = tc * w

    im = _gelu(_ln(x, ln_i_scale[0], ln_i_bias[0]) @ W_i[...] + b_i[0]) + x[:, :D]

    logits = lax.dot_general(im, wt, (((1,), (1,)), ((), ())))
    logits = logits / (jnp.abs(temp_ref[0, 0]) + 1e-6)

    mx = jnp.max(logits, axis=0, keepdims=True)
    e = jnp.exp(logits - mx)
    s = jnp.sum(e, axis=0, keepdims=True)
    sm = e / s
    scores = jnp.max(sm, axis=1, keepdims=True)

    ssum = jnp.sum(scores)
    lp = jnp.log(scores / (ssum + 1e-6))
    ls = jnp.log(scores)
    lam = lam_ref[0, 0]

    lin = lax.broadcasted_iota(jnp.int32, (NI, 1), 0)
    lane16 = lax.broadcasted_iota(jnp.int32, (1, NUM_QUERY), 1)

    def argmax_first(v):
        m = jnp.max(v)
        return jnp.min(jnp.where(v == m, lin, NI))

    idx = argmax_first(lp + g)
    sel_vec = jnp.full((1, NUM_QUERY), idx, jnp.int32)
    M = jnp.where(lin == idx, -jnp.inf, jnp.zeros((NI, 1), jnp.float32))
    S = jnp.zeros((NI, 1), jnp.float32)

    for k in range(1, NUM_QUERY):
        r = jnp.sum(jnp.where(lin == idx, logits, 0.0), axis=0, keepdims=True)
        d2 = jnp.sum((logits - r) ** 2, axis=1, keepdims=True)
        S = S + jnp.sqrt(d2)
        comb = ls + lam * (S / float(k)) + M
        idx = argmax_first(comb)
        sel_vec = jnp.where(lane16 == k, idx, sel_vec)
        M = jnp.where(lin == idx, -jnp.inf, M)

    sel_ref[0] = sel_vec


@functools.partial(jax.jit, static_argnames=())
def _run(image_features, text_features, gumbel,
         ln_t_scale, ln_t_bias, W_t, b_t, ln_i_scale, ln_i_bias, W_i, b_i,
         W_tw1, b_tw1, W_tw2, b_tw2, temperature, diversity_lambda):
    full = lambda shape: pl.BlockSpec(shape, lambda b: (0,) * len(shape))
    grid_spec = pl.GridSpec(
        grid=(B,),
        in_specs=[
            pl.BlockSpec((1, NI, H), lambda b: (b, 0, 0)),
            pl.BlockSpec((1, NT, H), lambda b: (b, 0, 0)),
            pl.BlockSpec((1, NI, 1), lambda b: (b, 0, 0)),
            full((1, H)), full((1, H)), full((H, D)), full((1, D)),
            full((1, H)), full((1, H)), full((H, D)), full((1, D)),
            full((H, D)), full((1, D)), full((D, 1)), full((1, 1)),
            pl.BlockSpec(memory_space=pltpu.SMEM),
            pl.BlockSpec(memory_space=pltpu.SMEM),
        ],
        out_specs=pl.BlockSpec((1, 1, NUM_QUERY), lambda b: (b, 0, 0)),
    )
    return pl.pallas_call(
        _select_body,
        grid_spec=grid_spec,
        out_shape=jax.ShapeDtypeStruct((B, 1, NUM_QUERY), jnp.int32),
    )(image_features, text_features, gumbel,
      ln_t_scale.reshape(1, H), ln_t_bias.reshape(1, H), W_t, b_t.reshape(1, D),
      ln_i_scale.reshape(1, H), ln_i_bias.reshape(1, H), W_i, b_i.reshape(1, D),
      W_tw1, b_tw1.reshape(1, D), W_tw2, b_tw2.reshape(1, 1),
      temperature.reshape(1, 1), diversity_lambda.reshape(1, 1))


def kernel(image_features, text_features, ln_t_scale, ln_t_bias, W_t, b_t,
           ln_i_scale, ln_i_bias, W_i, b_i, W_tw1, b_tw1, W_tw2, b_tw2,
           temperature, diversity_lambda):
    gum = jax.random.gumbel(jax.random.key(42), (B, NI), jnp.float32)
    sel = _run(image_features, text_features, gum.reshape(B, NI, 1),
               ln_t_scale, ln_t_bias, W_t, b_t, ln_i_scale, ln_i_bias, W_i,
               b_i, W_tw1, b_tw1, W_tw2, b_tw2,
               jnp.asarray(temperature, jnp.float32),
               jnp.asarray(diversity_lambda, jnp.float32))
    return sel.reshape(B, NUM_QUERY).astype(jnp.int64)


# speedup vs baseline: 2.0601x; 1.2523x over previous
import functools

import jax
import jax.numpy as jnp
from jax import lax
from jax.experimental import pallas as pl
from jax.experimental.pallas import tpu as pltpu

B, NI, NT, H, D = 16, 1024, 77, 768, 64
NTP = 80
NUM_QUERY = 16
_SQRT_HALF = 0.7071067811865476


def _gelu(x):
    return 0.5 * x * (1.0 + lax.erf(x * _SQRT_HALF))


def _ln(x, scale, bias):
    m = x.mean(-1, keepdims=True)
    v = ((x - m) ** 2).mean(-1, keepdims=True)
    inv = 1.0 / jnp.sqrt(v + 1e-5)
    return (x - m) * inv * scale + bias


def _select_body(img_ref, txt_ref, gum_ref,
                 ln_t_scale, ln_t_bias, W_t, b_t,
                 ln_i_scale, ln_i_bias, W_i, b_i,
                 W_tw1, b_tw1, W_tw2, b_tw2,
                 temp_ref, lam_ref, sel_ref):
    t = txt_ref[0]
    x = img_ref[0]
    g = gum_ref[0]

    tc = _gelu(_ln(t, ln_t_scale[0], ln_t_bias[0]) @ W_t[...] + b_t[0]) + t[:, :D]
    tw = _gelu(t @ W_tw1[...] + b_tw1[0]) @ W_tw2[...] + b_tw2[0]
    sg = jax.nn.sigmoid(tw)
    sg_m = sg - jnp.max(sg, axis=0, keepdims=True)
    e_t = jnp.exp(sg_m)
    w = e_t / jnp.sum(e_t, axis=0, keepdims=True)
    wt = tc * w
    wt = jnp.concatenate([wt, jnp.zeros((NTP - NT, D), jnp.float32)], axis=0)

    im = _gelu(_ln(x, ln_i_scale[0], ln_i_bias[0]) @ W_i[...] + b_i[0]) + x[:, :D]

    logits = lax.dot_general(wt, im, (((1,), (1,)), ((), ())))
    logits = logits / (jnp.abs(temp_ref[0, 0]) + 1e-6)

    row = lax.broadcasted_iota(jnp.int32, (NTP, 1), 0)
    mx = jnp.max(logits, axis=1, keepdims=True)
    e = jnp.where(row < NT, jnp.exp(logits - mx), 0.0)
    s = jnp.sum(e, axis=1, keepdims=True)
    s = jnp.where(row < NT, s, 1.0)
    sm = e / s
    scores = jnp.max(sm, axis=0, keepdims=True)

    ssum = jnp.sum(scores)
    lp = jnp.log(scores / (ssum + 1e-6))
    ls = jnp.log(scores)
    lam = lam_ref[0, 0]

    lin = lax.broadcasted_iota(jnp.int32, (1, NI), 1)
    lane16 = lax.broadcasted_iota(jnp.int32, (1, NUM_QUERY), 1)
    ones_row = jnp.ones((1, NTP), jnp.float32)

    def argmax_first(v):
        m = jnp.max(v)
        return jnp.min(jnp.where(v == m, lin, NI))

    idx = argmax_first(lp + g)
    sel_vec = jnp.full((1, NUM_QUERY), idx, jnp.int32)
    M = jnp.where(lin == idx, -jnp.inf, jnp.zeros((1, NI), jnp.float32))
    S = jnp.zeros((1, NI), jnp.float32)

    for k in range(1, NUM_QUERY):
        oh = (lin == idx).astype(jnp.float32)
        r = lax.dot_general(logits, oh, (((1,), (1,)), ((), ())))
        diff2 = (logits - r) ** 2
        d2 = lax.dot_general(ones_row, diff2, (((1,), (0,)), ((), ())))
        S = S + jnp.sqrt(d2)
        comb = ls + lam * (S / float(k)) + M
        idx = argmax_first(comb)
        sel_vec = jnp.where(lane16 == k, idx, sel_vec)
        M = jnp.where(lin == idx, -jnp.inf, M)

    sel_ref[0] = sel_vec


@functools.partial(jax.jit, static_argnames=())
def _run(image_features, text_features, gumbel,
         ln_t_scale, ln_t_bias, W_t, b_t, ln_i_scale, ln_i_bias, W_i, b_i,
         W_tw1, b_tw1, W_tw2, b_tw2, temperature, diversity_lambda):
    full = lambda shape: pl.BlockSpec(shape, lambda b: (0,) * len(shape))
    grid_spec = pl.GridSpec(
        grid=(B,),
        in_specs=[
            pl.BlockSpec((1, NI, H), lambda b: (b, 0, 0)),
            pl.BlockSpec((1, NT, H), lambda b: (b, 0, 0)),
            pl.BlockSpec((1, 1, NI), lambda b: (b, 0, 0)),
            full((1, H)), full((1, H)), full((H, D)), full((1, D)),
            full((1, H)), full((1, H)), full((H, D)), full((1, D)),
            full((H, D)), full((1, D)), full((D, 1)), full((1, 1)),
            pl.BlockSpec(memory_space=pltpu.SMEM),
            pl.BlockSpec(memory_space=pltpu.SMEM),
        ],
        out_specs=pl.BlockSpec((1, 1, NUM_QUERY), lambda b: (b, 0, 0)),
    )
    return pl.pallas_call(
        _select_body,
        grid_spec=grid_spec,
        out_shape=jax.ShapeDtypeStruct((B, 1, NUM_QUERY), jnp.int32),
    )(image_features, text_features, gumbel,
      ln_t_scale.reshape(1, H), ln_t_bias.reshape(1, H), W_t, b_t.reshape(1, D),
      ln_i_scale.reshape(1, H), ln_i_bias.reshape(1, H), W_i, b_i.reshape(1, D),
      W_tw1, b_tw1.reshape(1, D), W_tw2, b_tw2.reshape(1, 1),
      temperature.reshape(1, 1), diversity_lambda.reshape(1, 1))


def kernel(image_features, text_features, ln_t_scale, ln_t_bias, W_t, b_t,
           ln_i_scale, ln_i_bias, W_i, b_i, W_tw1, b_tw1, W_tw2, b_tw2,
           temperature, diversity_lambda):
    gum = jax.random.gumbel(jax.random.key(42), (B, NI), jnp.float32)
    sel = _run(image_features, text_features, gum.reshape(B, 1, NI),
               ln_t_scale, ln_t_bias, W_t, b_t, ln_i_scale, ln_i_bias, W_i,
               b_i, W_tw1, b_tw1, W_tw2, b_tw2,
               jnp.asarray(temperature, jnp.float32),
               jnp.asarray(diversity_lambda, jnp.float32))
    return sel.reshape(B, NUM_QUERY).astype(jnp.int64)


# speedup vs baseline: 2.0702x; 1.0049x over previous
import functools

import jax
import jax.numpy as jnp
from jax import lax
from jax.experimental import pallas as pl
from jax.experimental.pallas import tpu as pltpu

B, NI, NT, H, D = 16, 1024, 77, 768, 64
NTP = 80
NUM_QUERY = 16
BS = 2
_SQRT_HALF = 0.7071067811865476


def _gelu(x):
    return 0.5 * x * (1.0 + lax.erf(x * _SQRT_HALF))


def _ln(x, scale, bias):
    m = x.mean(-1, keepdims=True)
    v = ((x - m) ** 2).mean(-1, keepdims=True)
    inv = 1.0 / jnp.sqrt(v + 1e-5)
    return (x - m) * inv * scale + bias


def _select_body(img_ref, txt_ref, gum_ref,
                 ln_t_scale, ln_t_bias, W_t, b_t,
                 ln_i_scale, ln_i_bias, W_i, b_i,
                 W_tw1, b_tw1, W_tw2, b_tw2,
                 temp_ref, lam_ref, sel_ref):
  for j in range(BS):
    t = txt_ref[j]
    x = img_ref[j]
    g = gum_ref[j]

    tc = _gelu(_ln(t, ln_t_scale[0], ln_t_bias[0]) @ W_t[...] + b_t[0]) + t[:, :D]
    tw = _gelu(t @ W_tw1[...] + b_tw1[0]) @ W_tw2[...] + b_tw2[0]
    sg = jax.nn.sigmoid(tw)
    sg_m = sg - jnp.max(sg, axis=0, keepdims=True)
    e_t = jnp.exp(sg_m)
    w = e_t / jnp.sum(e_t, axis=0, keepdims=True)
    wt = tc * w
    wt = jnp.concatenate([wt, jnp.zeros((NTP - NT, D), jnp.float32)], axis=0)

    im = _gelu(_ln(x, ln_i_scale[0], ln_i_bias[0]) @ W_i[...] + b_i[0]) + x[:, :D]

    logits = lax.dot_general(wt, im, (((1,), (1,)), ((), ())))
    logits = logits / (jnp.abs(temp_ref[0, 0]) + 1e-6)

    row = lax.broadcasted_iota(jnp.int32, (NTP, 1), 0)
    mx = jnp.max(logits, axis=1, keepdims=True)
    e = jnp.where(row < NT, jnp.exp(logits - mx), 0.0)
    s = jnp.sum(e, axis=1, keepdims=True)
    s = jnp.where(row < NT, s, 1.0)
    sm = e / s
    scores = jnp.max(sm, axis=0, keepdims=True)

    ssum = jnp.sum(scores)
    lp = jnp.log(scores / (ssum + 1e-6))
    ls = jnp.log(scores)
    lam = lam_ref[0, 0]

    lin = lax.broadcasted_iota(jnp.int32, (1, NI), 1)
    lane16 = lax.broadcasted_iota(jnp.int32, (1, NUM_QUERY), 1)
    ones_row = jnp.ones((1, NTP), jnp.float32)

    def argmax_first(v):
        m = jnp.max(v)
        return jnp.min(jnp.where(v == m, lin, NI))

    idx = argmax_first(lp + g)
    sel_vec = jnp.full((1, NUM_QUERY), idx, jnp.int32)
    M = jnp.where(lin == idx, -jnp.inf, jnp.zeros((1, NI), jnp.float32))
    S = jnp.zeros((1, NI), jnp.float32)

    for k in range(1, NUM_QUERY):
        oh = (lin == idx).astype(jnp.float32)
        r = lax.dot_general(logits, oh, (((1,), (1,)), ((), ())))
        diff2 = (logits - r) ** 2
        d2 = lax.dot_general(ones_row, diff2, (((1,), (0,)), ((), ())))
        S = S + jnp.sqrt(d2)
        comb = ls + lam * (S / float(k)) + M
        idx = argmax_first(comb)
        sel_vec = jnp.where(lane16 == k, idx, sel_vec)
        M = jnp.where(lin == idx, -jnp.inf, M)

    sel_ref[j] = sel_vec


@functools.partial(jax.jit, static_argnames=())
def _run(image_features, text_features, gumbel,
         ln_t_scale, ln_t_bias, W_t, b_t, ln_i_scale, ln_i_bias, W_i, b_i,
         W_tw1, b_tw1, W_tw2, b_tw2, temperature, diversity_lambda):
    full = lambda shape: pl.BlockSpec(shape, lambda b: (0,) * len(shape))
    grid_spec = pl.GridSpec(
        grid=(B // BS,),
        in_specs=[
            pl.BlockSpec((BS, NI, H), lambda b: (b, 0, 0)),
            pl.BlockSpec((BS, NT, H), lambda b: (b, 0, 0)),
            pl.BlockSpec((BS, 1, NI), lambda b: (b, 0, 0)),
            full((1, H)), full((1, H)), full((H, D)), full((1, D)),
            full((1, H)), full((1, H)), full((H, D)), full((1, D)),
            full((H, D)), full((1, D)), full((D, 1)), full((1, 1)),
            pl.BlockSpec(memory_space=pltpu.SMEM),
            pl.BlockSpec(memory_space=pltpu.SMEM),
        ],
        out_specs=pl.BlockSpec((BS, 1, NUM_QUERY), lambda b: (b, 0, 0)),
    )
    return pl.pallas_call(
        _select_body,
        grid_spec=grid_spec,
        out_shape=jax.ShapeDtypeStruct((B, 1, NUM_QUERY), jnp.int32),
    )(image_features, text_features, gumbel,
      ln_t_scale.reshape(1, H), ln_t_bias.reshape(1, H), W_t, b_t.reshape(1, D),
      ln_i_scale.reshape(1, H), ln_i_bias.reshape(1, H), W_i, b_i.reshape(1, D),
      W_tw1, b_tw1.reshape(1, D), W_tw2, b_tw2.reshape(1, 1),
      temperature.reshape(1, 1), diversity_lambda.reshape(1, 1))


def kernel(image_features, text_features, ln_t_scale, ln_t_bias, W_t, b_t,
           ln_i_scale, ln_i_bias, W_i, b_i, W_tw1, b_tw1, W_tw2, b_tw2,
           temperature, diversity_lambda):
    gum = jax.random.gumbel(jax.random.key(42), (B, NI), jnp.float32)
    sel = _run(image_features, text_features, gum.reshape(B, 1, NI),
               ln_t_scale, ln_t_bias, W_t, b_t, ln_i_scale, ln_i_bias, W_i,
               b_i, W_tw1, b_tw1, W_tw2, b_tw2,
               jnp.asarray(temperature, jnp.float32),
               jnp.asarray(diversity_lambda, jnp.float32))
    return sel.reshape(B, NUM_QUERY).astype(jnp.int64)
